# baseline (speedup 1.0000x reference)
import jax
import jax.numpy as jnp
from jax.experimental import pallas as pl

EMA_DECAY = 0.99
K = 1024
D = 64
TN = 512


def _stage_a(x_ref, embT_ref, idx_ref, acc_ref):
    i = pl.program_id(0)
    x_tile = x_ref[...]
    embT = embT_ref[...]
    e2 = jnp.sum(embT * embT, axis=0, keepdims=True)
    a2 = jnp.sum(x_tile * x_tile, axis=1, keepdims=True)
    mm = jax.lax.dot_general(x_tile, embT, (((1,), (0,)), ((), ())),
                             preferred_element_type=jnp.float32)
    dist = (a2 - 2.0 * mm) + e2
    m = jnp.min(dist, axis=1, keepdims=True)
    iota = jax.lax.broadcasted_iota(jnp.int32, (TN, K), 1)
    cand = jnp.where(dist == m, iota, K)
    idxc = jnp.min(cand, axis=1, keepdims=True)
    idx_ref[...] = idxc

    onehot = (iota == idxc).astype(jnp.float32)
    xa = jnp.concatenate(
        [x_tile, jnp.ones((TN, 1), jnp.float32)], axis=1)
    acc = jax.lax.dot_general(onehot, xa, (((0,), (0,)), ((), ())),
                              preferred_element_type=jnp.float32,
                              precision=jax.lax.Precision.HIGHEST)

    @pl.when(i == 0)
    def _init():
        acc_ref[...] = acc

    @pl.when(i > 0)
    def _accum():
        acc_ref[...] = acc_ref[...] + acc


def _stage_b(acc_ref, cnt_ref, sumw_ref, emb_ref, xr_ref, istrain_ref,
             idx_ref, x_ref, out_ref):
    seg_sum = acc_ref[:, :D]
    seg_cnt = acc_ref[:, D:D + 1]
    cnt = cnt_ref[...]
    nc = cnt + (1.0 - EMA_DECAY) * (seg_cnt - cnt)
    usage = (nc >= 1.0).astype(jnp.float32)
    nc = nc * usage + (1.0 - usage)
    ns = sumw_ref[...] + (1.0 - EMA_DECAY) * (seg_sum - sumw_ref[...])
    emb_train = ns / nc
    emb_train = usage * emb_train + (1.0 - usage) * xr_ref[...]
    t = istrain_ref[0, 0] > 0.0
    emb = jnp.where(t, emb_train, emb_ref[...])

    idxc = idx_ref[...]
    iota = jax.lax.broadcasted_iota(jnp.int32, (TN, K), 1)
    onehot = (iota == idxc).astype(jnp.float32)
    q = jax.lax.dot_general(onehot, emb, (((1,), (0,)), ((), ())),
                            preferred_element_type=jnp.float32,
                            precision=jax.lax.Precision.HIGHEST)
    x_tile = x_ref[...]
    out_ref[...] = jnp.where(t, x_tile + (q - x_tile), q)


def kernel(x, is_training, embedding, count, sum_w):
    n = x.shape[0] * x.shape[1]
    flat = x.reshape(n, D)
    grid = n // TN
    embT = embedding.T

    idx, acc = pl.pallas_call(
        _stage_a,
        grid=(grid,),
        in_specs=[
            pl.BlockSpec((TN, D), lambda i: (i, 0)),
            pl.BlockSpec((D, K), lambda i: (0, 0)),
        ],
        out_specs=[
            pl.BlockSpec((TN, 1), lambda i: (i, 0)),
            pl.BlockSpec((K, D + 1), lambda i: (0, 0)),
        ],
        out_shape=[
            jax.ShapeDtypeStruct((n, 1), jnp.int32),
            jax.ShapeDtypeStruct((K, D + 1), jnp.float32),
        ],
    )(flat, embT)

    r = jax.random.randint(jax.random.key(42), (K,), 0, n)
    xr = jnp.take(flat, r, axis=0)
    istrain = jnp.asarray(is_training, jnp.float32).reshape(1, 1)
    cnt_col = count.reshape(K, 1)

    out = pl.pallas_call(
        _stage_b,
        grid=(grid,),
        in_specs=[
            pl.BlockSpec((K, D + 1), lambda i: (0, 0)),
            pl.BlockSpec((K, 1), lambda i: (0, 0)),
            pl.BlockSpec((K, D), lambda i: (0, 0)),
            pl.BlockSpec((K, D), lambda i: (0, 0)),
            pl.BlockSpec((K, D), lambda i: (0, 0)),
            pl.BlockSpec((1, 1), lambda i: (0, 0)),
            pl.BlockSpec((TN, 1), lambda i: (i, 0)),
            pl.BlockSpec((TN, D), lambda i: (i, 0)),
        ],
        out_specs=pl.BlockSpec((TN, D), lambda i: (i, 0)),
        out_shape=jax.ShapeDtypeStruct((n, D), jnp.float32),
    )(acc, cnt_col, sum_w, embedding, xr, istrain, idx, flat)

    return out.reshape(x.shape)


# speedup vs baseline: 2.0771x; 2.0771x over previous
import functools

import jax
import jax.numpy as jnp
from jax import lax
from jax.experimental import pallas as pl
from jax.experimental.pallas import tpu as pltpu
from jax.experimental.pallas import tpu_sc as plsc

EMA_DECAY = 0.99
K = 1024
D = 64
TN = 512

_SC_INFO = plsc.get_sparse_core_info()
NC = _SC_INFO.num_cores
NS = _SC_INFO.num_subcores
NW = NC * NS


def _stage_a(x_ref, embT_ref, idx_ref):
    x_tile = x_ref[...]
    embT = embT_ref[...]
    e2 = jnp.sum(embT * embT, axis=0, keepdims=True)
    a2 = jnp.sum(x_tile * x_tile, axis=1, keepdims=True)
    mm = jax.lax.dot_general(x_tile, embT, (((1,), (0,)), ((), ())),
                             preferred_element_type=jnp.float32)
    dist = (a2 - 2.0 * mm) + e2
    m = jnp.min(dist, axis=1, keepdims=True)
    iota = jax.lax.broadcasted_iota(jnp.int32, (TN, K), 1)
    cand = jnp.where(dist == m, iota, K)
    idx_ref[...] = jnp.min(cand, axis=1, keepdims=True)


def _sc_accum(flat_hbm, idx_hbm, r_hbm, z64_hbm, z8_hbm, ones8_hbm,
              psum_hbm, pcnt_hbm, xr_hbm,
              idx_v, rows_v, ones_v, ridx_v, xrrows_v, sem,
              ssum, scnt):
    c = lax.axis_index("c")
    s = lax.axis_index("s")
    wid = s * NC + c
    rows_per_w = flat_hbm.shape[0] // NW
    base = wid * rows_per_w

    @pl.when(s == 0)
    def _init():
        pltpu.sync_copy(z64_hbm, ssum)
        pltpu.sync_copy(z8_hbm, scnt)

    plsc.subcore_barrier()

    pltpu.sync_copy(idx_hbm.at[pl.ds(base, rows_per_w)], idx_v)
    pltpu.sync_copy(flat_hbm.at[pl.ds(base, rows_per_w)], rows_v)
    pltpu.sync_copy(ones8_hbm, ones_v)
    pltpu.sync_copy(rows_v, ssum.at[idx_v], add=True)
    pltpu.sync_copy(ones_v, scnt.at[idx_v], add=True)

    rpw = K // NW
    pltpu.sync_copy(r_hbm.at[pl.ds(wid * rpw, rpw)], ridx_v)
    pltpu.async_copy(flat_hbm.at[ridx_v], xrrows_v, sem).wait()
    pltpu.sync_copy(xrrows_v, xr_hbm.at[pl.ds(wid * rpw, rpw)])

    plsc.subcore_barrier()

    @pl.when(s == 0)
    def _publish():
        pltpu.sync_copy(ssum, psum_hbm.at[c])
        pltpu.sync_copy(scnt, pcnt_hbm.at[c])


def _stage_b(psum_ref, pcnt_ref, cnt_ref, sumw_ref, emb_ref, xr_ref,
             istrain_ref, out_ref):
    seg_sum = psum_ref[0] + psum_ref[1]
    seg_cnt = pcnt_ref[0, :, 0:1] + pcnt_ref[1, :, 0:1]
    cnt = cnt_ref[...]
    nc = cnt + (1.0 - EMA_DECAY) * (seg_cnt - cnt)
    usage = (nc >= 1.0).astype(jnp.float32)
    nc = nc * usage + (1.0 - usage)
    ns = sumw_ref[...] + (1.0 - EMA_DECAY) * (seg_sum - sumw_ref[...])
    emb_train = ns / nc
    emb_train = usage * emb_train + (1.0 - usage) * xr_ref[...]
    t = istrain_ref[0, 0] > 0.0
    out_ref[...] = jnp.where(t, emb_train, emb_ref[...])


def _sc_gather(emb_hbm, idx_hbm, out_hbm, idx_v, rows_v, sem):
    c = lax.axis_index("c")
    s = lax.axis_index("s")
    wid = s * NC + c
    rows_per_w = out_hbm.shape[0] // NW
    base = wid * rows_per_w
    pltpu.sync_copy(idx_hbm.at[pl.ds(base, rows_per_w)], idx_v)
    pltpu.async_copy(emb_hbm.at[idx_v], rows_v, sem).wait()
    pltpu.sync_copy(rows_v, out_hbm.at[pl.ds(base, rows_per_w)])


def kernel(x, is_training, embedding, count, sum_w):
    n = x.shape[0] * x.shape[1]
    flat = x.reshape(n, D)
    grid = n // TN
    embT = embedding.T
    rows_per_w = n // NW
    rpw = K // NW

    idx = pl.pallas_call(
        _stage_a,
        grid=(grid,),
        in_specs=[
            pl.BlockSpec((TN, D), lambda i: (i, 0)),
            pl.BlockSpec((D, K), lambda i: (0, 0)),
        ],
        out_specs=pl.BlockSpec((TN, 1), lambda i: (i, 0)),
        out_shape=jax.ShapeDtypeStruct((n, 1), jnp.int32),
    )(flat, embT)
    idx_flat = idx.reshape(n)

    r = jax.random.randint(jax.random.key(42), (K,), 0, n)
    z64 = jnp.zeros((K, D), jnp.float32)
    z8 = jnp.zeros((K, 8), jnp.float32)
    ones8 = jnp.ones((rows_per_w, 8), jnp.float32)

    mesh = plsc.VectorSubcoreMesh(core_axis_name="c", subcore_axis_name="s")
    sc_params = pltpu.CompilerParams(use_tc_tiling_on_sc=False)
    psum, pcnt, xr = pl.kernel(
        _sc_accum,
        compiler_params=sc_params,
        out_type=[
            jax.ShapeDtypeStruct((NC, K, D), jnp.float32),
            jax.ShapeDtypeStruct((NC, K, 8), jnp.float32),
            jax.ShapeDtypeStruct((K, D), jnp.float32),
        ],
        mesh=mesh,
        scratch_types=[
            pltpu.VMEM((rows_per_w,), jnp.int32),
            pltpu.VMEM((rows_per_w, D), jnp.float32),
            pltpu.VMEM((rows_per_w, 8), jnp.float32),
            pltpu.VMEM((rpw,), jnp.int32),
            pltpu.VMEM((rpw, D), jnp.float32),
            pltpu.SemaphoreType.DMA,
            pltpu.VMEM_SHARED((K, D), jnp.float32),
            pltpu.VMEM_SHARED((K, 8), jnp.float32),
        ],
    )(flat, idx_flat, r, z64, z8, ones8)

    istrain = jnp.asarray(is_training, jnp.float32).reshape(1, 1)
    cnt_col = count.reshape(K, 1)

    emb = pl.pallas_call(
        _stage_b,
        in_specs=[
            pl.BlockSpec((NC, K, D), lambda: (0, 0, 0)),
            pl.BlockSpec((NC, K, 8), lambda: (0, 0, 0)),
            pl.BlockSpec((K, 1), lambda: (0, 0)),
            pl.BlockSpec((K, D), lambda: (0, 0)),
            pl.BlockSpec((K, D), lambda: (0, 0)),
            pl.BlockSpec((K, D), lambda: (0, 0)),
            pl.BlockSpec((1, 1), lambda: (0, 0)),
        ],
        out_specs=pl.BlockSpec((K, D), lambda: (0, 0)),
        out_shape=jax.ShapeDtypeStruct((K, D), jnp.float32),
    )(psum, pcnt, cnt_col, sum_w, embedding, xr, istrain)

    out = pl.kernel(
        _sc_gather,
        compiler_params=sc_params,
        out_type=jax.ShapeDtypeStruct((n, D), jnp.float32),
        mesh=plsc.VectorSubcoreMesh(core_axis_name="c", subcore_axis_name="s"),
        scratch_types=[
            pltpu.VMEM((rows_per_w,), jnp.int32),
            pltpu.VMEM((rows_per_w, D), jnp.float32),
            pltpu.SemaphoreType.DMA,
        ],
    )(emb, idx_flat)

    return out.reshape(x.shape)
